# baseline (speedup 1.0000x reference)
import functools
import math

import jax
import jax.numpy as jnp
from jax.experimental import pallas as pl
from jax.experimental.pallas import tpu as pltpu


def _ceil_to(x, m):
    return (x + m - 1) // m * m


_WAVE = 256


def _dma_gather_kernel(idx_ref, emb_hbm, out_hbm, buf, rsems, wsem,
                       *, tokens_per_core, wave, scale):
    core = pl.program_id(0)
    tbase = core * tokens_per_core
    n_waves = tokens_per_core // wave

    def issue_read(w):
        sem = rsems.at[w % 4]
        for r in range(wave):
            tok = w * wave + r
            pltpu.make_async_copy(
                emb_hbm.at[pl.ds(idx_ref[tbase + tok], 1), :],
                buf.at[pl.ds(tok, 1), 0, :],
                sem,
            ).start(priority=r % 2)

    def drain_and_write(w):
        off = w * wave
        pltpu.make_async_copy(
            emb_hbm.at[pl.ds(0, wave), :],
            buf.at[pl.ds(off, wave), 0, :],
            rsems.at[w % 4],
        ).wait()
        buf[pl.ds(off, wave)] = buf[pl.ds(off, wave)] * scale
        pltpu.make_async_copy(
            buf.at[pl.ds(off, wave), 0, :],
            out_hbm.at[pl.ds(tbase + off, wave), :],
            wsem,
        ).start()

    lookahead = 3
    for w in range(min(lookahead, n_waves)):
        issue_read(w)
    for w in range(n_waves):
        if w + lookahead < n_waves:
            issue_read(w + lookahead)
        drain_and_write(w)
    pltpu.make_async_copy(
        buf.at[pl.ds(0, tokens_per_core), 0, :],
        out_hbm.at[pl.ds(tbase, tokens_per_core), :],
        wsem,
    ).wait()


def kernel(indices, emb_table):
    b, s = indices.shape
    v, d = emb_table.shape
    n = b * s
    scale = math.sqrt(float(d))

    flat_idx = jnp.clip(indices.reshape(n).astype(jnp.int32), 0, v - 1)

    wave = min(_WAVE, n)
    n_pad = _ceil_to(n, 2 * wave)
    if n_pad != n:
        flat_idx = jnp.pad(flat_idx, (0, n_pad - n))
    tokens_per_core = n_pad // 2

    out = pl.pallas_call(
        functools.partial(_dma_gather_kernel, tokens_per_core=tokens_per_core,
                          wave=wave, scale=scale),
        out_shape=jax.ShapeDtypeStruct((n_pad, d), emb_table.dtype),
        grid_spec=pltpu.PrefetchScalarGridSpec(
            num_scalar_prefetch=1,
            grid=(2,),
            in_specs=[pl.BlockSpec(memory_space=pl.ANY)],
            out_specs=pl.BlockSpec(memory_space=pl.ANY),
            scratch_shapes=[
                pltpu.VMEM((tokens_per_core, 1, d), emb_table.dtype),
                pltpu.SemaphoreType.DMA((4,)),
                pltpu.SemaphoreType.DMA,
            ],
        ),
        compiler_params=pltpu.CompilerParams(
            dimension_semantics=("parallel",),
            vmem_limit_bytes=40 << 20,
        ),
    )(flat_idx, emb_table)

    return out[:n].reshape(b, s, d)


# speedup vs baseline: 1.0156x; 1.0156x over previous
import functools
import math

import jax
import jax.numpy as jnp
from jax.experimental import pallas as pl
from jax.experimental.pallas import tpu as pltpu


def _ceil_to(x, m):
    return (x + m - 1) // m * m


_WAVE = 256


def _dma_gather_kernel(idx_ref, emb_hbm, out_hbm, buf, rsems, wsem,
                       *, tokens_per_core, wave, scale):
    core = pl.program_id(0)
    tbase = core * tokens_per_core
    n_waves = tokens_per_core // wave

    def issue_read(w):
        sem = rsems.at[w % 4]
        for r in range(wave):
            tok = w * wave + r
            pltpu.make_async_copy(
                emb_hbm.at[pl.ds(idx_ref[tbase + tok], 1), :],
                buf.at[pl.ds(tok, 1), :],
                sem,
            ).start(priority=r % 2)

    def drain_and_write(w):
        off = w * wave
        pltpu.make_async_copy(
            emb_hbm.at[pl.ds(0, wave), :],
            buf.at[pl.ds(off, wave), :],
            rsems.at[w % 4],
        ).wait()
        buf[pl.ds(off, wave), :] = buf[pl.ds(off, wave), :] * scale
        pltpu.make_async_copy(
            buf.at[pl.ds(off, wave), :],
            out_hbm.at[pl.ds(tbase + off, wave), :],
            wsem,
        ).start()

    lookahead = 3
    for w in range(min(lookahead, n_waves)):
        issue_read(w)
    for w in range(n_waves):
        if w + lookahead < n_waves:
            issue_read(w + lookahead)
        drain_and_write(w)
    pltpu.make_async_copy(
        buf.at[pl.ds(0, tokens_per_core), :],
        out_hbm.at[pl.ds(tbase, tokens_per_core), :],
        wsem,
    ).wait()


def kernel(indices, emb_table):
    b, s = indices.shape
    v, d = emb_table.shape
    n = b * s
    scale = math.sqrt(float(d))

    flat_idx = jnp.clip(indices.reshape(n).astype(jnp.int32), 0, v - 1)

    wave = min(_WAVE, n)
    n_pad = _ceil_to(n, 2 * wave)
    if n_pad != n:
        flat_idx = jnp.pad(flat_idx, (0, n_pad - n))
    tokens_per_core = n_pad // 2

    out = pl.pallas_call(
        functools.partial(_dma_gather_kernel, tokens_per_core=tokens_per_core,
                          wave=wave, scale=scale),
        out_shape=jax.ShapeDtypeStruct((n_pad, d), emb_table.dtype),
        grid_spec=pltpu.PrefetchScalarGridSpec(
            num_scalar_prefetch=1,
            grid=(2,),
            in_specs=[pl.BlockSpec(memory_space=pl.ANY)],
            out_specs=pl.BlockSpec(memory_space=pl.ANY),
            scratch_shapes=[
                pltpu.VMEM((tokens_per_core, d), emb_table.dtype),
                pltpu.SemaphoreType.DMA((4,)),
                pltpu.SemaphoreType.DMA,
            ],
        ),
        compiler_params=pltpu.CompilerParams(
            dimension_semantics=("parallel",),
            vmem_limit_bytes=40 << 20,
        ),
    )(flat_idx, emb_table)

    return out[:n].reshape(b, s, d)
